# Initial kernel scaffold; baseline (speedup 1.0000x reference)
import jax
import jax.numpy as jnp
from jax.experimental import pallas as pl


def kernel(feats, points, W1, b1, g1, beta1, W2, b2, g2, beta2):
    raise NotImplementedError("write your pallas kernel here")


# speedup vs baseline: 11.9067x; 11.9067x over previous
import functools

import jax
import jax.numpy as jnp
from jax import lax
from jax.experimental import pallas as pl
from jax.experimental.pallas import tpu as pltpu
from jax.experimental.pallas import tpu_sc as plsc

_B, _N, _S, _K = 4, 4096, 1024, 16
_CIN, _COUT = 128, 256
_NT = 2048
_RS = 256
_EPS = 1e-5
_NW = 32
_WROWS = (_B * _S) // _NW
_CHUNK = 8


def _mlp_body(f_ref, w1_ref, w2_ref, s1_ref, t1_ref, s2_ref, t2_ref, o_ref):
    f = f_ref[0]
    y = lax.dot_general(w1_ref[...], f, (((1,), (0,)), ((), ())),
                        preferred_element_type=jnp.float32,
                        precision=lax.Precision.HIGHEST)
    y = jnp.maximum(y * s1_ref[...] + t1_ref[...], 0.0)
    zt = lax.dot_general(y, w2_ref[...], (((0,), (1,)), ((), ())),
                         preferred_element_type=jnp.float32,
                         precision=lax.Precision.HIGHEST)
    o_ref[...] = jnp.maximum(zt * s2_ref[...] + t2_ref[...], 0.0)


def _mlp(feats, W1, W2, s1, t1, s2, t2):
    return pl.pallas_call(
        _mlp_body,
        grid=(_B, _N // _NT),
        in_specs=[
            pl.BlockSpec((1, _CIN, _NT), lambda b, j: (b, 0, j)),
            pl.BlockSpec((_COUT, _CIN), lambda b, j: (0, 0)),
            pl.BlockSpec((_COUT, _COUT), lambda b, j: (0, 0)),
            pl.BlockSpec((_COUT, 1), lambda b, j: (0, 0)),
            pl.BlockSpec((_COUT, 1), lambda b, j: (0, 0)),
            pl.BlockSpec((1, _COUT), lambda b, j: (0, 0)),
            pl.BlockSpec((1, _COUT), lambda b, j: (0, 0)),
        ],
        out_specs=pl.BlockSpec((_NT, _COUT), lambda b, j: (b * (_N // _NT) + j, 0)),
        out_shape=jax.ShapeDtypeStruct((_B * _N, _COUT), jnp.float32),
    )(feats, W1, W2, s1, t1, s2, t2)


def _fps_body(px_ref, py_ref, pz_ref, cx_ref, cy_ref, cz_ref):
    px, py, pz = px_ref[...], py_ref[...], pz_ref[...]
    iota = lax.broadcasted_iota(jnp.int32, (_B, _N), 1)
    iota_s = lax.broadcasted_iota(jnp.int32, (_B, _S), 1)

    def body(i, carry):
        dists, far = carry
        onehot = iota == far
        cxv = jnp.sum(jnp.where(onehot, px, 0.0), axis=1, keepdims=True)
        cyv = jnp.sum(jnp.where(onehot, py, 0.0), axis=1, keepdims=True)
        czv = jnp.sum(jnp.where(onehot, pz, 0.0), axis=1, keepdims=True)
        sel = iota_s == i
        cx_ref[...] = jnp.where(sel, cxv, cx_ref[...])
        cy_ref[...] = jnp.where(sel, cyv, cy_ref[...])
        cz_ref[...] = jnp.where(sel, czv, cz_ref[...])
        d = (px - cxv) ** 2 + (py - cyv) ** 2 + (pz - czv) ** 2
        dists = jnp.minimum(dists, d)
        m = jnp.max(dists, axis=1, keepdims=True)
        far = jnp.min(jnp.where(dists == m, iota, _N), axis=1, keepdims=True)
        return dists, far

    cx_ref[...] = jnp.zeros((_B, _S), jnp.float32)
    cy_ref[...] = jnp.zeros((_B, _S), jnp.float32)
    cz_ref[...] = jnp.zeros((_B, _S), jnp.float32)
    lax.fori_loop(
        0, _S, body,
        (jnp.full((_B, _N), 1e10, jnp.float32), jnp.zeros((_B, 1), jnp.int32)))


def _fps(px, py, pz):
    return pl.pallas_call(
        _fps_body,
        out_shape=[jax.ShapeDtypeStruct((_B, _S), jnp.float32)] * 3,
    )(px, py, pz)


def _knn_body(px_ref, py_ref, pz_ref, cx_ref, cy_ref, cz_ref, idx_ref, d2_ref):
    b = pl.program_id(0)
    px, py, pz = px_ref[0], py_ref[0], pz_ref[0]
    cx, cy, cz = cx_ref[0], cy_ref[0], cz_ref[0]
    d2_ref[...] = (px - cx) ** 2 + (py - cy) ** 2 + (pz - cz) ** 2
    iota = lax.broadcasted_iota(jnp.int32, (_RS, _N), 1)
    base = b * _N
    for k in range(_K):
        d2c = d2_ref[...]
        m = jnp.min(d2c, axis=1, keepdims=True)
        j = jnp.min(jnp.where(d2c == m, iota, _N), axis=1, keepdims=True)
        idx_ref[0, :, pl.ds(k, 1)] = j + base
        if k != _K - 1:
            d2_ref[...] = jnp.where(iota == j, jnp.float32(jnp.inf), d2c)


def _knn(px, py, pz, cx, cy, cz):
    pspec = pl.BlockSpec((1, 1, _N), lambda b, j: (b, 0, 0))
    cspec = pl.BlockSpec((1, _RS, 1), lambda b, j: (b, j, 0))
    return pl.pallas_call(
        _knn_body,
        grid=(_B, _S // _RS),
        in_specs=[pspec, pspec, pspec, cspec, cspec, cspec],
        out_specs=pl.BlockSpec((1, _RS, _K), lambda b, j: (b, j, 0)),
        out_shape=jax.ShapeDtypeStruct((_B, _S, _K), jnp.int32),
        scratch_shapes=[pltpu.VMEM((_RS, _N), jnp.float32)],
    )(px[:, None, :], py[:, None, :], pz[:, None, :], cx, cy, cz)


def _pool_body(x_hbm, idx_hbm, o_hbm, idx_v, rows_v, out_v, sem):
    wid = lax.axis_index("s") * 2 + lax.axis_index("c")
    base = wid * _WROWS
    pltpu.sync_copy(idx_hbm.at[pl.ds(base * _K, _WROWS * _K)], idx_v)

    def chunk_body(ch, _):
        cp = pltpu.async_copy(
            x_hbm.at[idx_v.at[pl.ds(ch * (_CHUNK * _K), _CHUNK * _K)]],
            rows_v, sem)
        cp.wait()

        def row_body(r, _):
            for cc in range(_COUT // 16):
                sl = pl.ds(cc * 16, 16)
                acc = rows_v[r * _K, sl]
                for t in range(1, _K):
                    acc = acc + rows_v[r * _K + t, sl]
                out_v[ch * _CHUNK + r, sl] = acc * (1.0 / _K)
            return 0

        lax.fori_loop(0, _CHUNK, row_body, 0)
        return 0

    lax.fori_loop(0, _WROWS // _CHUNK, chunk_body, 0)
    pltpu.sync_copy(out_v, o_hbm.at[pl.ds(base, _WROWS)])


def _pool(xt, idx_flat):
    f = pl.kernel(
        _pool_body,
        out_type=jax.ShapeDtypeStruct((_B * _S, _COUT), jnp.float32),
        mesh=plsc.VectorSubcoreMesh(core_axis_name="c", subcore_axis_name="s"),
        scratch_types=[
            pltpu.VMEM((_WROWS * _K,), jnp.int32),
            pltpu.VMEM((_CHUNK * _K, _COUT), jnp.float32),
            pltpu.VMEM((_WROWS, _COUT), jnp.float32),
            pltpu.SemaphoreType.DMA,
        ],
    )
    return f(xt, idx_flat)


def kernel(feats, points, W1, b1, g1, beta1, W2, b2, g2, beta2):
    s1 = g1 / jnp.sqrt(1.0 + _EPS)
    t1 = s1 * b1 + beta1
    s2 = g2 / jnp.sqrt(1.0 + _EPS)
    t2 = s2 * b2 + beta2
    xt = _mlp(feats, W1, W2, s1[:, None], t1[:, None], s2[None, :], t2[None, :])
    px, py, pz = points[:, 0, :], points[:, 1, :], points[:, 2, :]
    cx, cy, cz = _fps(px, py, pz)
    centroids = jnp.stack([cx, cy, cz], axis=1)
    idx = _knn(px, py, pz, cx[:, :, None], cy[:, :, None], cz[:, :, None])
    outt = _pool(xt, idx.reshape(-1))
    out = outt.reshape(_B, _S, _COUT).transpose(0, 2, 1)
    return (out, centroids)
